# baseline (speedup 1.0000x reference)
import jax
import jax.numpy as jnp
import numpy as np
from jax import lax
from jax.experimental import pallas as pl
from jax.experimental.pallas import tpu as pltpu
from jax.experimental.pallas import tpu_sc as plsc

N = 10000
E = 320000
D = 128
A = 10
R = 8
H = 64

NC = 2
NS = 16
L = 16
D2 = D // NC

CHUNK = 80
EPS = E // NS
CPS = EPS // CHUNK
NBUF = 3
NROUND = CPS // NBUF

ACC_W = D2 + L
N_PAD = 10240
ROWS_PER_TEC = N_PAD // NS

INV_SQRT_R = float(1.0 / np.sqrt(R))
INV_SQRT_H = float(1.0 / np.sqrt(H))
INV_SQRT_D = float(1.0 / np.sqrt(D))
INV_SQRT_DA = float(1.0 / np.sqrt(D * A))


def _edge_body(eft_ref, eat_ref, w0_ref, w1_ref, w2_ref, w3_ref, wd_ref,
               y_ref, dt_ref):
    eft = eft_ref[...]
    h = jax.nn.silu(lax.dot_general(eft, w0_ref[...], (((0,), (0,)), ((), ())),
                                    preferred_element_type=jnp.float32)
                    * INV_SQRT_R)
    h = jax.nn.silu(jnp.dot(h, w1_ref[...], preferred_element_type=jnp.float32)
                    * INV_SQRT_H)
    h = jax.nn.silu(jnp.dot(h, w2_ref[...], preferred_element_type=jnp.float32)
                    * INV_SQRT_H)
    tp = jnp.dot(h, w3_ref[...], preferred_element_type=jnp.float32) * INV_SQRT_H
    y_ref[...] = tp
    dt = lax.dot_general(wd_ref[...], eft, (((0,), (0,)), ((), ())),
                         preferred_element_type=jnp.float32) * INV_SQRT_R
    dt_ref[0:1, :] = jnp.tanh(dt * dt)
    dt_ref[1:2, :] = eat_ref[...]


def _edge_stage(eft, eat, w0, w1, w2, w3, wd):
    be = 6400
    return pl.pallas_call(
        _edge_body,
        grid=(E // be,),
        in_specs=[
            pl.BlockSpec((R, be), lambda i: (0, i)),
            pl.BlockSpec((1, be), lambda i: (0, i)),
            pl.BlockSpec((R, H), lambda i: (0, 0)),
            pl.BlockSpec((H, H), lambda i: (0, 0)),
            pl.BlockSpec((H, H), lambda i: (0, 0)),
            pl.BlockSpec((H, D), lambda i: (0, 0)),
            pl.BlockSpec((R, 1), lambda i: (0, 0)),
        ],
        out_specs=[
            pl.BlockSpec((be, D), lambda i: (i, 0)),
            pl.BlockSpec((2, be), lambda i: (0, i)),
        ],
        out_shape=[
            jax.ShapeDtypeStruct((E, D), jnp.float32),
            jax.ShapeDtypeStruct((2, E), jnp.float32),
        ],
    )(eft, eat, w0, w1, w2, w3, wd)


def _node_body(nf_ref, wup_ref, x2_ref):
    x = jnp.dot(nf_ref[...], wup_ref[...],
                preferred_element_type=jnp.float32) * INV_SQRT_D
    x2_ref[0] = x[:, :D2]
    x2_ref[1] = x[:, D2:]


def _node_stage(nf, wup):
    bn = 2000
    return pl.pallas_call(
        _node_body,
        grid=(N // bn,),
        in_specs=[
            pl.BlockSpec((bn, D), lambda i: (i, 0)),
            pl.BlockSpec((D, D), lambda i: (0, 0)),
        ],
        out_specs=pl.BlockSpec((NC, bn, D2), lambda i: (0, i, 0)),
        out_shape=jax.ShapeDtypeStruct((NC, N, D2), jnp.float32),
    )(nf, wup)


def _sc_body(x2, y_hbm, dens_hbm, aux_hbm, out_hbm, acc,
             aux_v, y_v, xg_v, m_v, d_v, ridx_s,
             sem_x, sem_y, sem_d, sem_g, sem_o):
    cid = lax.axis_index("c")
    sid = lax.axis_index("s")

    def zrow(r, _):
        for c in range(ACC_W // L):
            m_v[0, r, pl.ds(c * L, L)] = jnp.zeros((L,), jnp.float32)
        return 0
    lax.fori_loop(0, CHUNK, zrow, 0)
    for z in range(ROWS_PER_TEC // CHUNK):
        pltpu.sync_copy(m_v.at[0],
                        acc.at[pl.ds(sid * ROWS_PER_TEC + z * CHUNK, CHUNK)])
    plsc.subcore_barrier()

    def pipeline(c):
        xh = x2.at[c]
        col0 = c * D2

        def issue_in(g, b):
            base = sid * EPS + g * CHUNK
            pltpu.async_copy(aux_hbm.at[sid * CPS + g], aux_v.at[b],
                             sem_x.at[b])
            pltpu.async_copy(y_hbm.at[pl.ds(base, CHUNK), pl.ds(col0, D2)],
                             y_v.at[b], sem_y.at[b])
            pltpu.async_copy(dens_hbm.at[pl.ds(0, 2), pl.ds(base, CHUNK)],
                             d_v.at[b], sem_d.at[b])

        def wait_aux(b):
            pltpu.make_async_copy(aux_hbm.at[0], aux_v.at[b],
                                  sem_x.at[b]).wait()

        def issue_gather(b):
            pltpu.async_copy(xh.at[aux_v.at[b, 0]], xg_v.at[b], sem_g.at[b])

        def wait_rest(b):
            pltpu.make_async_copy(y_hbm.at[pl.ds(0, CHUNK), pl.ds(col0, D2)],
                                  y_v.at[b], sem_y.at[b]).wait()
            pltpu.make_async_copy(dens_hbm.at[pl.ds(0, 2), pl.ds(0, CHUNK)],
                                  d_v.at[b], sem_d.at[b]).wait()
            pltpu.make_async_copy(xh.at[aux_v.at[b, 0]], xg_v.at[b],
                                  sem_g.at[b]).wait()

        def wait_scatter(b):
            pltpu.make_async_copy(m_v.at[b], acc.at[ridx_s.at[b]],
                                  sem_o.at[b]).wait()

        def step(g, b):
            @pl.when(g + 1 < CPS)
            def _():
                wait_aux((b + 1) % NBUF)
                issue_gather((b + 1) % NBUF)

            wait_rest(b)

            @pl.when(g >= NBUF)
            def _():
                wait_scatter(b)

            @plsc.parallel_loop(0, CHUNK // L)
            def _(g16):
                d16 = d_v[b, 0, pl.ds(g16 * L, L)]
                a16 = d_v[b, 1, pl.ds(g16 * L, L)]
                ridx_s[b, pl.ds(g16 * L, L)] = aux_v[b, 1, pl.ds(g16 * L, L)]
                for j in range(L):
                    e = g16 * L + j
                    asp = jnp.full((L,), a16[j], jnp.float32)
                    for k in range(D2 // L):
                        m_v[b, e, pl.ds(k * L, L)] = (
                            xg_v[b, e, pl.ds(k * L, L)]
                            * y_v[b, e, pl.ds(k * L, L)] * asp)
                    m_v[b, e, pl.ds(D2, L)] = jnp.full((L,), d16[j],
                                                       jnp.float32)

            pltpu.async_copy(m_v.at[b], acc.at[ridx_s.at[b]], sem_o.at[b],
                             add=True)

            @pl.when(g + NBUF < CPS)
            def _():
                issue_in(g + NBUF, b)

        for b in range(NBUF):
            issue_in(b, b)
        wait_aux(0)
        issue_gather(0)

        def round_body(p, _):
            for b in range(NBUF):
                step(p * NBUF + b, b)
            return 0
        lax.fori_loop(0, NROUND, round_body, 0)
        for t in range(CPS % NBUF):
            step(jnp.int32(NROUND * NBUF + t), t)
        for b in range(NBUF):
            wait_scatter(b)
        plsc.subcore_barrier()

        for z in range(ROWS_PER_TEC // CHUNK):
            row0 = sid * ROWS_PER_TEC + z * CHUNK
            pltpu.sync_copy(acc.at[pl.ds(row0, CHUNK)], m_v.at[0])
            pltpu.sync_copy(m_v.at[0], out_hbm.at[c, pl.ds(row0, CHUNK)])

    pipeline(cid)


def _sc_stage(x2, y, dens, aux):
    mesh = plsc.VectorSubcoreMesh(core_axis_name="c", subcore_axis_name="s")
    fn = pl.kernel(
        _sc_body,
        out_type=jax.ShapeDtypeStruct((NC, N_PAD, ACC_W), jnp.float32),
        mesh=mesh,
        scratch_types=[
            pltpu.VMEM_SHARED((N_PAD, ACC_W), jnp.float32),
            pltpu.VMEM((NBUF, 2, CHUNK), jnp.int32),
            pltpu.VMEM((NBUF, CHUNK, D2), jnp.float32),
            pltpu.VMEM((NBUF, CHUNK, D2), jnp.float32),
            pltpu.VMEM((NBUF, CHUNK, ACC_W), jnp.float32),
            pltpu.VMEM((NBUF, 2, CHUNK), jnp.float32),
            pltpu.VMEM((NBUF, CHUNK), jnp.int32),
            pltpu.SemaphoreType.DMA((NBUF,)),
            pltpu.SemaphoreType.DMA((NBUF,)),
            pltpu.SemaphoreType.DMA((NBUF,)),
            pltpu.SemaphoreType.DMA((NBUF,)),
            pltpu.SemaphoreType.DMA((NBUF,)),
        ],
        compiler_params=pltpu.CompilerParams(use_tc_tiling_on_sc=False),
    )
    return fn(x2, y, dens, aux)


def _final_body(nf_ref, na_ref, part_ref, wl_ref, wsk_ref, out_ref, sc_ref):
    p0 = part_ref[0]
    p1 = part_ref[1]
    msg = jnp.concatenate([p0[:, :D2], p1[:, :D2]], axis=1)
    dens = p0[:, D2:D2 + 1]
    outm = (jnp.dot(msg, wl_ref[...], preferred_element_type=jnp.float32)
            * INV_SQRT_D) / (dens + 1.0)
    out_ref[...] = outm

    nf = nf_ref[...]
    na = na_ref[...]
    wsk = wsk_ref[...]
    acc = jnp.zeros_like(nf)
    for v in range(A):
        acc = acc + jnp.dot(nf * na[:, v:v + 1], wsk[:, v, :],
                            preferred_element_type=jnp.float32)
    sc_ref[...] = acc * INV_SQRT_DA


def _final_stage(nf, na, partials, wl, wsk):
    bn = 1000
    return pl.pallas_call(
        _final_body,
        grid=(N // bn,),
        in_specs=[
            pl.BlockSpec((bn, D), lambda i: (i, 0)),
            pl.BlockSpec((bn, A), lambda i: (i, 0)),
            pl.BlockSpec((NC, bn, ACC_W), lambda i: (0, i, 0)),
            pl.BlockSpec((D, D), lambda i: (0, 0)),
            pl.BlockSpec((D, A, D), lambda i: (0, 0, 0)),
        ],
        out_specs=[
            pl.BlockSpec((bn, D), lambda i: (i, 0)),
            pl.BlockSpec((bn, D), lambda i: (i, 0)),
        ],
        out_shape=[
            jax.ShapeDtypeStruct((N, D), jnp.float32),
            jax.ShapeDtypeStruct((N, D), jnp.float32),
        ],
    )(nf, na, partials, wl, wsk)


def kernel(node_attrs, node_feats, edge_attrs, edge_feats, edge_index,
           W_up, W0, W1, W2, W3, Wd, Wl, Wsk):
    sender = edge_index[0].astype(jnp.int32)
    receiver = edge_index[1].astype(jnp.int32)
    eft = edge_feats.T
    eat = edge_attrs.reshape(1, E)
    aux = jnp.stack([sender.reshape(-1, CHUNK), receiver.reshape(-1, CHUNK)],
                    axis=1)

    y, dens_t = _edge_stage(eft, eat, W0, W1, W2, W3, Wd)
    x2 = _node_stage(node_feats, W_up)
    partials = _sc_stage(x2, y, dens_t, aux)
    out, sc = _final_stage(node_feats, node_attrs, partials, Wl, Wsk)
    return out.reshape(N, D, 1), sc


# speedup vs baseline: 1.1687x; 1.1687x over previous
import jax
import jax.numpy as jnp
import numpy as np
from jax import lax
from jax.experimental import pallas as pl
from jax.experimental.pallas import tpu as pltpu
from jax.experimental.pallas import tpu_sc as plsc

N = 10000
E = 320000
D = 128
A = 10
R = 8
H = 64

NC = 2
NS = 16
L = 16
D2 = D // NC

CHUNK = 80
EPS = E // NS
CPS = EPS // CHUNK
NBUF = 3
NROUND = CPS // NBUF

ACC_W = D2 + L
N_PAD = 10240
ROWS_PER_TEC = N_PAD // NS

INV_SQRT_R = float(1.0 / np.sqrt(R))
INV_SQRT_H = float(1.0 / np.sqrt(H))
INV_SQRT_D = float(1.0 / np.sqrt(D))
INV_SQRT_DA = float(1.0 / np.sqrt(D * A))


def _edge_body(eft_ref, eat_ref, w0_ref, w1_ref, w2_ref, w3_ref, wd_ref,
               y_ref, dt_ref):
    eft = eft_ref[...]
    h = jax.nn.silu(lax.dot_general(eft, w0_ref[...], (((0,), (0,)), ((), ())),
                                    preferred_element_type=jnp.float32)
                    * INV_SQRT_R)
    h = jax.nn.silu(jnp.dot(h, w1_ref[...], preferred_element_type=jnp.float32)
                    * INV_SQRT_H)
    h = jax.nn.silu(jnp.dot(h, w2_ref[...], preferred_element_type=jnp.float32)
                    * INV_SQRT_H)
    tp = jnp.dot(h, w3_ref[...], preferred_element_type=jnp.float32) * INV_SQRT_H
    y_ref[...] = tp
    dt = lax.dot_general(wd_ref[...], eft, (((0,), (0,)), ((), ())),
                         preferred_element_type=jnp.float32) * INV_SQRT_R
    dt_ref[0:1, :] = jnp.tanh(dt * dt)
    dt_ref[1:2, :] = eat_ref[...]


def _edge_stage(eft, eat, w0, w1, w2, w3, wd):
    be = 6400
    return pl.pallas_call(
        _edge_body,
        grid=(E // be,),
        in_specs=[
            pl.BlockSpec((R, be), lambda i: (0, i)),
            pl.BlockSpec((1, be), lambda i: (0, i)),
            pl.BlockSpec((R, H), lambda i: (0, 0)),
            pl.BlockSpec((H, H), lambda i: (0, 0)),
            pl.BlockSpec((H, H), lambda i: (0, 0)),
            pl.BlockSpec((H, D), lambda i: (0, 0)),
            pl.BlockSpec((R, 1), lambda i: (0, 0)),
        ],
        out_specs=[
            pl.BlockSpec((be, D), lambda i: (i, 0)),
            pl.BlockSpec((2, be), lambda i: (0, i)),
        ],
        out_shape=[
            jax.ShapeDtypeStruct((E, D), jnp.float32),
            jax.ShapeDtypeStruct((2, E), jnp.float32),
        ],
    )(eft, eat, w0, w1, w2, w3, wd)


def _node_body(nf_ref, wup_ref, x2_ref):
    x = jnp.dot(nf_ref[...], wup_ref[...],
                preferred_element_type=jnp.float32) * INV_SQRT_D
    x2_ref[0] = x[:, :D2]
    x2_ref[1] = x[:, D2:]


def _node_stage(nf, wup):
    bn = 2000
    return pl.pallas_call(
        _node_body,
        grid=(N // bn,),
        in_specs=[
            pl.BlockSpec((bn, D), lambda i: (i, 0)),
            pl.BlockSpec((D, D), lambda i: (0, 0)),
        ],
        out_specs=pl.BlockSpec((NC, bn, D2), lambda i: (0, i, 0)),
        out_shape=jax.ShapeDtypeStruct((NC, N, D2), jnp.float32),
    )(nf, wup)


def _sc_body(x2, y_hbm, dens_hbm, aux_hbm, out_hbm, acc,
             aux_v, y_v, xg_v, m_v, d_v, ridx_s,
             sem_x, sem_y, sem_d, sem_g, sem_o):
    cid = lax.axis_index("c")
    sid = lax.axis_index("s")

    def zrow(r, _):
        for c in range(ACC_W // L):
            m_v[0, r, pl.ds(c * L, L)] = jnp.zeros((L,), jnp.float32)
        return 0
    lax.fori_loop(0, CHUNK, zrow, 0)
    for z in range(ROWS_PER_TEC // CHUNK):
        pltpu.sync_copy(m_v.at[0],
                        acc.at[pl.ds(sid * ROWS_PER_TEC + z * CHUNK, CHUNK)])
    plsc.subcore_barrier()

    def pipeline(c):
        xh = x2.at[c]
        col0 = c * D2

        def issue_in(g, b):
            base = sid * EPS + g * CHUNK
            pltpu.async_copy(aux_hbm.at[sid * CPS + g], aux_v.at[b],
                             sem_x.at[b])
            pltpu.async_copy(y_hbm.at[pl.ds(base, CHUNK), pl.ds(col0, D2)],
                             y_v.at[b], sem_y.at[b])
            pltpu.async_copy(dens_hbm.at[pl.ds(0, 2), pl.ds(base, CHUNK)],
                             d_v.at[b], sem_d.at[b])

        def wait_aux(b):
            pltpu.make_async_copy(aux_hbm.at[0], aux_v.at[b],
                                  sem_x.at[b]).wait()

        def issue_gather(b):
            pltpu.async_copy(xh.at[aux_v.at[b, 0]], xg_v.at[b], sem_g.at[b])

        def wait_rest(b):
            pltpu.make_async_copy(y_hbm.at[pl.ds(0, CHUNK), pl.ds(col0, D2)],
                                  y_v.at[b], sem_y.at[b]).wait()
            pltpu.make_async_copy(dens_hbm.at[pl.ds(0, 2), pl.ds(0, CHUNK)],
                                  d_v.at[b], sem_d.at[b]).wait()
            pltpu.make_async_copy(xh.at[aux_v.at[b, 0]], xg_v.at[b],
                                  sem_g.at[b]).wait()

        def wait_scatter(b):
            pltpu.make_async_copy(m_v.at[b], acc.at[ridx_s.at[b]],
                                  sem_o.at[b]).wait()

        def step(g, b):
            @pl.when(g + 1 < CPS)
            def _():
                wait_aux((b + 1) % NBUF)
                issue_gather((b + 1) % NBUF)

            wait_rest(b)

            @pl.when(g >= NBUF)
            def _():
                wait_scatter(b)

            @plsc.parallel_loop(0, CHUNK // L, unroll=5)
            def _(g16):
                d16 = d_v[b, 0, pl.ds(g16 * L, L)]
                a16 = d_v[b, 1, pl.ds(g16 * L, L)]
                ridx_s[b, pl.ds(g16 * L, L)] = aux_v[b, 1, pl.ds(g16 * L, L)]
                for j in range(L):
                    e = g16 * L + j
                    asp = jnp.full((L,), a16[j], jnp.float32)
                    for k in range(D2 // L):
                        m_v[b, e, pl.ds(k * L, L)] = (
                            xg_v[b, e, pl.ds(k * L, L)]
                            * y_v[b, e, pl.ds(k * L, L)] * asp)
                    m_v[b, e, pl.ds(D2, L)] = jnp.full((L,), d16[j],
                                                       jnp.float32)

            pltpu.async_copy(m_v.at[b], acc.at[ridx_s.at[b]], sem_o.at[b],
                             add=True)

            @pl.when(g + NBUF < CPS)
            def _():
                issue_in(g + NBUF, b)

        for b in range(NBUF):
            issue_in(b, b)
        wait_aux(0)
        issue_gather(0)

        def round_body(p, _):
            for b in range(NBUF):
                step(p * NBUF + b, b)
            return 0
        lax.fori_loop(0, NROUND, round_body, 0)
        for t in range(CPS % NBUF):
            step(jnp.int32(NROUND * NBUF + t), t)
        for b in range(NBUF):
            wait_scatter(b)
        plsc.subcore_barrier()

        for z in range(ROWS_PER_TEC // CHUNK):
            row0 = sid * ROWS_PER_TEC + z * CHUNK
            pltpu.sync_copy(acc.at[pl.ds(row0, CHUNK)], m_v.at[0])
            pltpu.sync_copy(m_v.at[0], out_hbm.at[c, pl.ds(row0, CHUNK)])

    pipeline(cid)


def _sc_stage(x2, y, dens, aux):
    mesh = plsc.VectorSubcoreMesh(core_axis_name="c", subcore_axis_name="s")
    fn = pl.kernel(
        _sc_body,
        out_type=jax.ShapeDtypeStruct((NC, N_PAD, ACC_W), jnp.float32),
        mesh=mesh,
        scratch_types=[
            pltpu.VMEM_SHARED((N_PAD, ACC_W), jnp.float32),
            pltpu.VMEM((NBUF, 2, CHUNK), jnp.int32),
            pltpu.VMEM((NBUF, CHUNK, D2), jnp.float32),
            pltpu.VMEM((NBUF, CHUNK, D2), jnp.float32),
            pltpu.VMEM((NBUF, CHUNK, ACC_W), jnp.float32),
            pltpu.VMEM((NBUF, 2, CHUNK), jnp.float32),
            pltpu.VMEM((NBUF, CHUNK), jnp.int32),
            pltpu.SemaphoreType.DMA((NBUF,)),
            pltpu.SemaphoreType.DMA((NBUF,)),
            pltpu.SemaphoreType.DMA((NBUF,)),
            pltpu.SemaphoreType.DMA((NBUF,)),
            pltpu.SemaphoreType.DMA((NBUF,)),
        ],
        compiler_params=pltpu.CompilerParams(use_tc_tiling_on_sc=False),
    )
    return fn(x2, y, dens, aux)


def _final_body(nf_ref, na_ref, part_ref, wl_ref, wsk_ref, out_ref, sc_ref):
    p0 = part_ref[0]
    p1 = part_ref[1]
    msg = jnp.concatenate([p0[:, :D2], p1[:, :D2]], axis=1)
    dens = p0[:, D2:D2 + 1]
    outm = (jnp.dot(msg, wl_ref[...], preferred_element_type=jnp.float32)
            * INV_SQRT_D) / (dens + 1.0)
    out_ref[...] = outm

    nf = nf_ref[...]
    na = na_ref[...]
    wsk = wsk_ref[...]
    acc = jnp.zeros_like(nf)
    for v in range(A):
        acc = acc + jnp.dot(nf * na[:, v:v + 1], wsk[:, v, :],
                            preferred_element_type=jnp.float32)
    sc_ref[...] = acc * INV_SQRT_DA


def _final_stage(nf, na, partials, wl, wsk):
    bn = 1000
    return pl.pallas_call(
        _final_body,
        grid=(N // bn,),
        in_specs=[
            pl.BlockSpec((bn, D), lambda i: (i, 0)),
            pl.BlockSpec((bn, A), lambda i: (i, 0)),
            pl.BlockSpec((NC, bn, ACC_W), lambda i: (0, i, 0)),
            pl.BlockSpec((D, D), lambda i: (0, 0)),
            pl.BlockSpec((D, A, D), lambda i: (0, 0, 0)),
        ],
        out_specs=[
            pl.BlockSpec((bn, D), lambda i: (i, 0)),
            pl.BlockSpec((bn, D), lambda i: (i, 0)),
        ],
        out_shape=[
            jax.ShapeDtypeStruct((N, D), jnp.float32),
            jax.ShapeDtypeStruct((N, D), jnp.float32),
        ],
    )(nf, na, partials, wl, wsk)


def kernel(node_attrs, node_feats, edge_attrs, edge_feats, edge_index,
           W_up, W0, W1, W2, W3, Wd, Wl, Wsk):
    sender = edge_index[0].astype(jnp.int32)
    receiver = edge_index[1].astype(jnp.int32)
    eft = edge_feats.T
    eat = edge_attrs.reshape(1, E)
    aux = jnp.stack([sender.reshape(-1, CHUNK), receiver.reshape(-1, CHUNK)],
                    axis=1)

    y, dens_t = _edge_stage(eft, eat, W0, W1, W2, W3, Wd)
    x2 = _node_stage(node_feats, W_up)
    partials = _sc_stage(x2, y, dens_t, aux)
    out, sc = _final_stage(node_feats, node_attrs, partials, Wl, Wsk)
    return out.reshape(N, D, 1), sc


# speedup vs baseline: 1.1769x; 1.0070x over previous
import jax
import jax.numpy as jnp
import numpy as np
from jax import lax
from jax.experimental import pallas as pl
from jax.experimental.pallas import tpu as pltpu
from jax.experimental.pallas import tpu_sc as plsc

N = 10000
E = 320000
D = 128
A = 10
R = 8
H = 64

NC = 2
NS = 16
L = 16
D2 = D // NC

CHUNK = 80
EPS = E // NS
CPS = EPS // CHUNK
NBUF = 3
NROUND = CPS // NBUF

ACC_W = D2 + L
N_PAD = 10240
ROWS_PER_TEC = N_PAD // NS

INV_SQRT_R = float(1.0 / np.sqrt(R))
INV_SQRT_H = float(1.0 / np.sqrt(H))
INV_SQRT_D = float(1.0 / np.sqrt(D))
INV_SQRT_DA = float(1.0 / np.sqrt(D * A))


def _edge_body(eft_ref, eat_ref, w0_ref, w1_ref, w2_ref, w3_ref, wd_ref,
               y_ref, dt_ref):
    eft = eft_ref[...]
    h = jax.nn.silu(lax.dot_general(eft, w0_ref[...], (((0,), (0,)), ((), ())),
                                    preferred_element_type=jnp.float32)
                    * INV_SQRT_R)
    b16 = jnp.bfloat16
    h = jax.nn.silu(jnp.dot(h.astype(b16), w1_ref[...].astype(b16),
                            preferred_element_type=jnp.float32) * INV_SQRT_H)
    h = jax.nn.silu(jnp.dot(h.astype(b16), w2_ref[...].astype(b16),
                            preferred_element_type=jnp.float32) * INV_SQRT_H)
    tp = jnp.dot(h.astype(b16), w3_ref[...].astype(b16),
                 preferred_element_type=jnp.float32) * INV_SQRT_H
    y_ref[...] = tp
    dt = lax.dot_general(wd_ref[...], eft, (((0,), (0,)), ((), ())),
                         preferred_element_type=jnp.float32) * INV_SQRT_R
    dt_ref[0:1, :] = jnp.tanh(dt * dt)
    dt_ref[1:2, :] = eat_ref[...]


def _edge_stage(eft, eat, w0, w1, w2, w3, wd):
    be = 6400
    return pl.pallas_call(
        _edge_body,
        grid=(E // be,),
        in_specs=[
            pl.BlockSpec((R, be), lambda i: (0, i)),
            pl.BlockSpec((1, be), lambda i: (0, i)),
            pl.BlockSpec((R, H), lambda i: (0, 0)),
            pl.BlockSpec((H, H), lambda i: (0, 0)),
            pl.BlockSpec((H, H), lambda i: (0, 0)),
            pl.BlockSpec((H, D), lambda i: (0, 0)),
            pl.BlockSpec((R, 1), lambda i: (0, 0)),
        ],
        out_specs=[
            pl.BlockSpec((be, D), lambda i: (i, 0)),
            pl.BlockSpec((2, be), lambda i: (0, i)),
        ],
        out_shape=[
            jax.ShapeDtypeStruct((E, D), jnp.float32),
            jax.ShapeDtypeStruct((2, E), jnp.float32),
        ],
    )(eft, eat, w0, w1, w2, w3, wd)


def _node_body(nf_ref, wup_ref, x2_ref):
    x = jnp.dot(nf_ref[...], wup_ref[...],
                preferred_element_type=jnp.float32) * INV_SQRT_D
    x2_ref[0] = x[:, :D2]
    x2_ref[1] = x[:, D2:]


def _node_stage(nf, wup):
    bn = 2000
    return pl.pallas_call(
        _node_body,
        grid=(N // bn,),
        in_specs=[
            pl.BlockSpec((bn, D), lambda i: (i, 0)),
            pl.BlockSpec((D, D), lambda i: (0, 0)),
        ],
        out_specs=pl.BlockSpec((NC, bn, D2), lambda i: (0, i, 0)),
        out_shape=jax.ShapeDtypeStruct((NC, N, D2), jnp.float32),
    )(nf, wup)


def _sc_body(x2, y_hbm, dens_hbm, aux_hbm, out_hbm, acc,
             aux_v, y_v, xg_v, m_v, d_v, ridx_s,
             sem_x, sem_y, sem_d, sem_g, sem_o):
    cid = lax.axis_index("c")
    sid = lax.axis_index("s")

    def zrow(r, _):
        for c in range(ACC_W // L):
            m_v[0, r, pl.ds(c * L, L)] = jnp.zeros((L,), jnp.float32)
        return 0
    lax.fori_loop(0, CHUNK, zrow, 0)
    for z in range(ROWS_PER_TEC // CHUNK):
        pltpu.sync_copy(m_v.at[0],
                        acc.at[pl.ds(sid * ROWS_PER_TEC + z * CHUNK, CHUNK)])
    plsc.subcore_barrier()

    def pipeline(c):
        xh = x2.at[c]
        col0 = c * D2

        def issue_in(g, b):
            base = sid * EPS + g * CHUNK
            pltpu.async_copy(aux_hbm.at[sid * CPS + g], aux_v.at[b],
                             sem_x.at[b])
            pltpu.async_copy(y_hbm.at[pl.ds(base, CHUNK), pl.ds(col0, D2)],
                             y_v.at[b], sem_y.at[b])
            pltpu.async_copy(dens_hbm.at[pl.ds(0, 2), pl.ds(base, CHUNK)],
                             d_v.at[b], sem_d.at[b])

        def wait_aux(b):
            pltpu.make_async_copy(aux_hbm.at[0], aux_v.at[b],
                                  sem_x.at[b]).wait()

        def issue_gather(b):
            pltpu.async_copy(xh.at[aux_v.at[b, 0]], xg_v.at[b], sem_g.at[b])

        def wait_rest(b):
            pltpu.make_async_copy(y_hbm.at[pl.ds(0, CHUNK), pl.ds(col0, D2)],
                                  y_v.at[b], sem_y.at[b]).wait()
            pltpu.make_async_copy(dens_hbm.at[pl.ds(0, 2), pl.ds(0, CHUNK)],
                                  d_v.at[b], sem_d.at[b]).wait()
            pltpu.make_async_copy(xh.at[aux_v.at[b, 0]], xg_v.at[b],
                                  sem_g.at[b]).wait()

        def wait_scatter(b):
            pltpu.make_async_copy(m_v.at[b], acc.at[ridx_s.at[b]],
                                  sem_o.at[b]).wait()

        def step(g, b):
            @pl.when(g + 1 < CPS)
            def _():
                wait_aux((b + 1) % NBUF)
                issue_gather((b + 1) % NBUF)

            wait_rest(b)

            @pl.when(g >= NBUF)
            def _():
                wait_scatter(b)

            @plsc.parallel_loop(0, CHUNK // L, unroll=5)
            def _(g16):
                d16 = d_v[b, 0, pl.ds(g16 * L, L)]
                a16 = d_v[b, 1, pl.ds(g16 * L, L)]
                ridx_s[b, pl.ds(g16 * L, L)] = aux_v[b, 1, pl.ds(g16 * L, L)]
                for j in range(L):
                    e = g16 * L + j
                    asp = jnp.full((L,), a16[j], jnp.float32)
                    for k in range(D2 // L):
                        m_v[b, e, pl.ds(k * L, L)] = (
                            xg_v[b, e, pl.ds(k * L, L)]
                            * y_v[b, e, pl.ds(k * L, L)] * asp)
                    m_v[b, e, pl.ds(D2, L)] = jnp.full((L,), d16[j],
                                                       jnp.float32)

            pltpu.async_copy(m_v.at[b], acc.at[ridx_s.at[b]], sem_o.at[b],
                             add=True)

            @pl.when(g + NBUF < CPS)
            def _():
                issue_in(g + NBUF, b)

        for b in range(NBUF):
            issue_in(b, b)
        wait_aux(0)
        issue_gather(0)

        def round_body(p, _):
            for b in range(NBUF):
                step(p * NBUF + b, b)
            return 0
        lax.fori_loop(0, NROUND, round_body, 0)
        for t in range(CPS % NBUF):
            step(jnp.int32(NROUND * NBUF + t), t)
        for b in range(NBUF):
            wait_scatter(b)
        plsc.subcore_barrier()

        for z in range(ROWS_PER_TEC // CHUNK):
            row0 = sid * ROWS_PER_TEC + z * CHUNK
            pltpu.sync_copy(acc.at[pl.ds(row0, CHUNK)], m_v.at[0])
            pltpu.sync_copy(m_v.at[0], out_hbm.at[c, pl.ds(row0, CHUNK)])

    pipeline(cid)


def _sc_stage(x2, y, dens, aux):
    mesh = plsc.VectorSubcoreMesh(core_axis_name="c", subcore_axis_name="s")
    fn = pl.kernel(
        _sc_body,
        out_type=jax.ShapeDtypeStruct((NC, N_PAD, ACC_W), jnp.float32),
        mesh=mesh,
        scratch_types=[
            pltpu.VMEM_SHARED((N_PAD, ACC_W), jnp.float32),
            pltpu.VMEM((NBUF, 2, CHUNK), jnp.int32),
            pltpu.VMEM((NBUF, CHUNK, D2), jnp.float32),
            pltpu.VMEM((NBUF, CHUNK, D2), jnp.float32),
            pltpu.VMEM((NBUF, CHUNK, ACC_W), jnp.float32),
            pltpu.VMEM((NBUF, 2, CHUNK), jnp.float32),
            pltpu.VMEM((NBUF, CHUNK), jnp.int32),
            pltpu.SemaphoreType.DMA((NBUF,)),
            pltpu.SemaphoreType.DMA((NBUF,)),
            pltpu.SemaphoreType.DMA((NBUF,)),
            pltpu.SemaphoreType.DMA((NBUF,)),
            pltpu.SemaphoreType.DMA((NBUF,)),
        ],
        compiler_params=pltpu.CompilerParams(use_tc_tiling_on_sc=False),
    )
    return fn(x2, y, dens, aux)


def _final_body(nf_ref, na_ref, part_ref, wl_ref, wsk_ref, out_ref, sc_ref):
    p0 = part_ref[0]
    p1 = part_ref[1]
    msg = jnp.concatenate([p0[:, :D2], p1[:, :D2]], axis=1)
    dens = p0[:, D2:D2 + 1]
    outm = (jnp.dot(msg, wl_ref[...], preferred_element_type=jnp.float32)
            * INV_SQRT_D) / (dens + 1.0)
    out_ref[...] = outm

    nf = nf_ref[...]
    na = na_ref[...]
    wsk = wsk_ref[...]
    acc = jnp.zeros_like(nf)
    for v in range(A):
        acc = acc + jnp.dot(nf * na[:, v:v + 1], wsk[:, v, :],
                            preferred_element_type=jnp.float32)
    sc_ref[...] = acc * INV_SQRT_DA


def _final_stage(nf, na, partials, wl, wsk):
    bn = 1000
    return pl.pallas_call(
        _final_body,
        grid=(N // bn,),
        in_specs=[
            pl.BlockSpec((bn, D), lambda i: (i, 0)),
            pl.BlockSpec((bn, A), lambda i: (i, 0)),
            pl.BlockSpec((NC, bn, ACC_W), lambda i: (0, i, 0)),
            pl.BlockSpec((D, D), lambda i: (0, 0)),
            pl.BlockSpec((D, A, D), lambda i: (0, 0, 0)),
        ],
        out_specs=[
            pl.BlockSpec((bn, D), lambda i: (i, 0)),
            pl.BlockSpec((bn, D), lambda i: (i, 0)),
        ],
        out_shape=[
            jax.ShapeDtypeStruct((N, D), jnp.float32),
            jax.ShapeDtypeStruct((N, D), jnp.float32),
        ],
    )(nf, na, partials, wl, wsk)


def kernel(node_attrs, node_feats, edge_attrs, edge_feats, edge_index,
           W_up, W0, W1, W2, W3, Wd, Wl, Wsk):
    sender = edge_index[0].astype(jnp.int32)
    receiver = edge_index[1].astype(jnp.int32)
    eft = edge_feats.T
    eat = edge_attrs.reshape(1, E)
    aux = jnp.stack([sender.reshape(-1, CHUNK), receiver.reshape(-1, CHUNK)],
                    axis=1)

    y, dens_t = _edge_stage(eft, eat, W0, W1, W2, W3, Wd)
    x2 = _node_stage(node_feats, W_up)
    partials = _sc_stage(x2, y, dens_t, aux)
    out, sc = _final_stage(node_feats, node_attrs, partials, Wl, Wsk)
    return out.reshape(N, D, 1), sc
